# Initial kernel scaffold; baseline (speedup 1.0000x reference)
import jax
import jax.numpy as jnp
from jax.experimental import pallas as pl


def kernel(x, Wg, bg, W1, b1, W2, b2):
    raise NotImplementedError("write your pallas kernel here")


# speedup vs baseline: 1.5187x; 1.5187x over previous
import jax
import jax.numpy as jnp
from jax import lax
from jax.experimental import pallas as pl
from jax.experimental.pallas import tpu as pltpu
from jax.experimental.pallas import tpu_sc as plsc

TOKENS, DIM, HID, E = 2048, 1024, 4096, 8
TM = 128
NT = 24
CAP = NT * TM
TH = 512
NH = HID // TH
NC, NS = 2, 16
NW = NC * NS
BPW = CAP // NW
OUT_PAD = TOKENS + CAP

_NEG = -1e30


def _gate_body(x_ref, wg_ref, bg_ref, top1_ref, prob_ref):
    l = jnp.dot(x_ref[...], wg_ref[...],
                preferred_element_type=jnp.float32) + bg_ref[...]
    m = jnp.max(l, axis=1, keepdims=True)
    s = jnp.sum(jnp.exp(l - m), axis=1, keepdims=True)
    col = lax.broadcasted_iota(jnp.int32, l.shape, 1)
    top1_ref[...] = jnp.min(jnp.where(l == m, col, 128), axis=1, keepdims=True)
    prob_ref[...] = 1.0 / s


def _gate(x, Wg, bg):
    wgp = jnp.pad(Wg, ((0, 0), (0, 128 - E)))
    bgp = jnp.pad(bg, (0, 128 - E), constant_values=_NEG)[None, :]
    return pl.pallas_call(
        _gate_body,
        out_shape=(jax.ShapeDtypeStruct((TOKENS, 1), jnp.int32),
                   jax.ShapeDtypeStruct((TOKENS, 1), jnp.float32)),
    )(x, wgp, bgp)


def _sc_gather_body(x_hbm, perm_hbm, xs_hbm, idx_v, rows_v, sem):
    wid = lax.axis_index("s") * NC + lax.axis_index("c")
    base = wid * BPW
    pltpu.sync_copy(perm_hbm.at[pl.ds(base, BPW)], idx_v)
    pltpu.async_copy(x_hbm.at[idx_v], rows_v, sem).wait()
    pltpu.sync_copy(rows_v, xs_hbm.at[pl.ds(base, BPW)])


def _gather_rows(x, perm):
    return pl.kernel(
        _sc_gather_body,
        out_type=jax.ShapeDtypeStruct((CAP, DIM), jnp.float32),
        mesh=plsc.VectorSubcoreMesh(core_axis_name="c", subcore_axis_name="s"),
        scratch_types=[pltpu.VMEM((BPW,), jnp.int32),
                       pltpu.VMEM((BPW, DIM), jnp.float32),
                       pltpu.SemaphoreType.DMA],
    )(x, perm)


def _sc_scatter_body(ys_hbm, scat_hbm, out_hbm, idx_v, rows_v, sem):
    wid = lax.axis_index("s") * NC + lax.axis_index("c")
    base = wid * BPW
    pltpu.sync_copy(scat_hbm.at[pl.ds(base, BPW)], idx_v)
    pltpu.sync_copy(ys_hbm.at[pl.ds(base, BPW)], rows_v)
    pltpu.async_copy(rows_v, out_hbm.at[idx_v], sem).wait()


def _scatter_rows(ys, scat):
    return pl.kernel(
        _sc_scatter_body,
        out_type=jax.ShapeDtypeStruct((OUT_PAD, DIM), jnp.float32),
        mesh=plsc.VectorSubcoreMesh(core_axis_name="c", subcore_axis_name="s"),
        scratch_types=[pltpu.VMEM((BPW,), jnp.int32),
                       pltpu.VMEM((BPW, DIM), jnp.float32),
                       pltpu.SemaphoreType.DMA],
    )(ys, scat)


def _ffn_body(ntl_ref, ts_ref, fe_ref, xs_ref, w1_ref, b1_ref, w2_ref, b2_ref,
              wv_ref, out_ref):
    del fe_ref
    e = pl.program_id(0)
    h = pl.program_id(1)
    ne = ntl_ref[e]
    ts = ts_ref[e]

    def tile_body(i, carry):
        base = (ts + i) * TM
        rows = pl.ds(base, TM)
        xt = xs_ref[rows, :]
        ht = jnp.maximum(
            jnp.dot(xt, w1_ref[0], preferred_element_type=jnp.float32)
            + b1_ref[0], 0.0)
        yt = jnp.dot(ht, w2_ref[0], preferred_element_type=jnp.float32)

        @pl.when(h == 0)
        def _():
            out_ref[rows, :] = yt

        @pl.when(h != 0)
        def _():
            out_ref[rows, :] = out_ref[rows, :] + yt

        @pl.when(h == NH - 1)
        def _():
            out_ref[rows, :] = ((out_ref[rows, :] + b2_ref[0])
                                * wv_ref[rows, :])

        return carry

    lax.fori_loop(0, ne, tile_body, 0)


def _grouped_ffn(xs, W1, b1, W2, b2, wcol, ntl, tstart, fetch_e):
    grid_spec = pltpu.PrefetchScalarGridSpec(
        num_scalar_prefetch=3,
        grid=(E, NH),
        in_specs=[
            pl.BlockSpec((CAP, DIM), lambda e, h, ntl, ts, fe: (0, 0)),
            pl.BlockSpec((1, DIM, TH), lambda e, h, ntl, ts, fe: (fe[e], 0, h)),
            pl.BlockSpec((1, 1, TH), lambda e, h, ntl, ts, fe: (fe[e], 0, h)),
            pl.BlockSpec((1, TH, DIM), lambda e, h, ntl, ts, fe: (fe[e], h, 0)),
            pl.BlockSpec((1, 1, DIM), lambda e, h, ntl, ts, fe: (fe[e], 0, 0)),
            pl.BlockSpec((CAP, 1), lambda e, h, ntl, ts, fe: (0, 0)),
        ],
        out_specs=pl.BlockSpec((CAP, DIM), lambda e, h, ntl, ts, fe: (0, 0)),
    )
    return pl.pallas_call(
        _ffn_body,
        grid_spec=grid_spec,
        out_shape=jax.ShapeDtypeStruct((CAP, DIM), jnp.float32),
        compiler_params=pltpu.CompilerParams(
            dimension_semantics=("arbitrary", "arbitrary")),
    )(ntl, tstart, fetch_e, xs, W1, b1[:, None, :], W2, b2[:, None, :], wcol)


def kernel(x, Wg, bg, W1, b1, W2, b2):
    i32 = jnp.int32
    top1_2d, prob_2d = _gate(x, Wg, bg)
    top1 = top1_2d[:, 0]
    gp = prob_2d[:, 0]

    oh = (top1[:, None] == jnp.arange(E, dtype=i32)[None, :]).astype(i32)
    cum = jnp.cumsum(oh, axis=0)
    rank = jnp.take_along_axis(cum, top1[:, None], axis=1)[:, 0] - 1
    counts = cum[-1]
    ntl = (counts + TM - 1) // TM
    tstart = jnp.concatenate([jnp.zeros((1,), i32),
                              jnp.cumsum(ntl)[:-1].astype(i32)])
    slot = tstart[top1] * TM + rank
    tok = jnp.arange(TOKENS, dtype=i32)
    perm = jnp.zeros((CAP,), i32).at[slot].set(tok)
    wcol = jnp.zeros((CAP, 1), jnp.float32).at[slot, 0].set(gp)
    scat = (TOKENS + jnp.arange(CAP, dtype=i32)).at[slot].set(tok)
    fe_raw = jnp.where(ntl > 0, jnp.arange(E, dtype=i32), -1)
    fetch_e = jnp.maximum(lax.cummax(fe_raw), 0).astype(i32)

    xs = _gather_rows(x, perm)
    ys = _grouped_ffn(xs, W1, b1, W2, b2, wcol,
                      ntl.astype(i32), tstart.astype(i32), fetch_e)
    out_big = _scatter_rows(ys, scat)
    return out_big[:TOKENS]


# speedup vs baseline: 2.3032x; 1.5166x over previous
import jax
import jax.numpy as jnp
from jax import lax
from jax.experimental import pallas as pl
from jax.experimental.pallas import tpu as pltpu
from jax.experimental.pallas import tpu_sc as plsc

TOKENS, DIM, HID, E = 2048, 1024, 4096, 8
TM = 128
NT = 24
CAP = NT * TM
TH = 1024
NH = HID // TH
NC, NS = 2, 16
NW = NC * NS
BPW = CAP // NW
BPW_IN = TOKENS // NW
OUT_PAD = TOKENS + CAP

_NEG = -1e30


def _gate_body(x_ref, wg_ref, bg_ref, top1_ref, prob_ref):
    l = jnp.dot(x_ref[...], wg_ref[...],
                preferred_element_type=jnp.float32) + bg_ref[...]
    m = jnp.max(l, axis=1, keepdims=True)
    s = jnp.sum(jnp.exp(l - m), axis=1, keepdims=True)
    col = lax.broadcasted_iota(jnp.int32, l.shape, 1)
    top1_ref[...] = jnp.min(jnp.where(l == m, col, 128), axis=1, keepdims=True)
    prob_ref[...] = 1.0 / s


def _gate(x, Wg, bg):
    wgp = jnp.pad(Wg, ((0, 0), (0, 128 - E)))
    bgp = jnp.pad(bg, (0, 128 - E), constant_values=_NEG)[None, :]
    return pl.pallas_call(
        _gate_body,
        out_shape=(jax.ShapeDtypeStruct((TOKENS, 1), jnp.int32),
                   jax.ShapeDtypeStruct((TOKENS, 1), jnp.float32)),
    )(x, wgp, bgp)


def _sc_scatter_in_body(x_hbm, slot_hbm, xs_hbm, idx_v, rows_v, sem):
    wid = lax.axis_index("s") * NC + lax.axis_index("c")
    base = wid * BPW_IN
    pltpu.sync_copy(slot_hbm.at[pl.ds(base, BPW_IN)], idx_v)
    pltpu.sync_copy(x_hbm.at[pl.ds(base, BPW_IN)], rows_v)
    pltpu.async_copy(rows_v, xs_hbm.at[idx_v], sem).wait()


def _dispatch_rows(x, slot):
    return pl.kernel(
        _sc_scatter_in_body,
        out_type=jax.ShapeDtypeStruct((CAP, DIM), jnp.float32),
        mesh=plsc.VectorSubcoreMesh(core_axis_name="c", subcore_axis_name="s"),
        scratch_types=[pltpu.VMEM((BPW_IN,), jnp.int32),
                       pltpu.VMEM((BPW_IN, DIM), jnp.float32),
                       pltpu.SemaphoreType.DMA],
    )(x, slot)


def _sc_scatter_body(ys_hbm, scat_hbm, out_hbm, idx_v, rows_v, sem):
    wid = lax.axis_index("s") * NC + lax.axis_index("c")
    base = wid * BPW
    pltpu.sync_copy(scat_hbm.at[pl.ds(base, BPW)], idx_v)
    pltpu.sync_copy(ys_hbm.at[pl.ds(base, BPW)], rows_v)
    pltpu.async_copy(rows_v, out_hbm.at[idx_v], sem).wait()


def _scatter_rows(ys, scat):
    return pl.kernel(
        _sc_scatter_body,
        out_type=jax.ShapeDtypeStruct((OUT_PAD, DIM), jnp.float32),
        mesh=plsc.VectorSubcoreMesh(core_axis_name="c", subcore_axis_name="s"),
        scratch_types=[pltpu.VMEM((BPW,), jnp.int32),
                       pltpu.VMEM((BPW, DIM), jnp.float32),
                       pltpu.SemaphoreType.DMA],
    )(ys, scat)


def _ffn_body(ntl_ref, ts_ref, fe_ref, xs_ref, w1_ref, b1_ref, w2_ref, b2_ref,
              wv_ref, out_ref):
    del fe_ref
    e = pl.program_id(0)
    h = pl.program_id(1)
    ne = ntl_ref[e]
    ts = ts_ref[e]

    def tile_body(i, carry):
        base = (ts + i) * TM
        rows = pl.ds(base, TM)
        xt = xs_ref[rows, :]
        ht = jnp.maximum(
            jnp.dot(xt, w1_ref[0], preferred_element_type=jnp.float32)
            + b1_ref[0], 0.0)
        yt = jnp.dot(ht, w2_ref[0], preferred_element_type=jnp.float32)

        @pl.when(h == 0)
        def _():
            out_ref[rows, :] = yt

        @pl.when(h != 0)
        def _():
            out_ref[rows, :] = out_ref[rows, :] + yt

        @pl.when(h == NH - 1)
        def _():
            out_ref[rows, :] = ((out_ref[rows, :] + b2_ref[0])
                                * wv_ref[rows, :])

        return carry

    lax.fori_loop(0, ne, tile_body, 0)


def _grouped_ffn(xs, W1, b1, W2, b2, wcol, ntl, tstart, fetch_e):
    grid_spec = pltpu.PrefetchScalarGridSpec(
        num_scalar_prefetch=3,
        grid=(E, NH),
        in_specs=[
            pl.BlockSpec((CAP, DIM), lambda e, h, ntl, ts, fe: (0, 0)),
            pl.BlockSpec((1, DIM, TH), lambda e, h, ntl, ts, fe: (fe[e], 0, h)),
            pl.BlockSpec((1, 1, TH), lambda e, h, ntl, ts, fe: (fe[e], 0, h)),
            pl.BlockSpec((1, TH, DIM), lambda e, h, ntl, ts, fe: (fe[e], h, 0)),
            pl.BlockSpec((1, 1, DIM), lambda e, h, ntl, ts, fe: (fe[e], 0, 0)),
            pl.BlockSpec((CAP, 1), lambda e, h, ntl, ts, fe: (0, 0)),
        ],
        out_specs=pl.BlockSpec((CAP, DIM), lambda e, h, ntl, ts, fe: (0, 0)),
    )
    return pl.pallas_call(
        _ffn_body,
        grid_spec=grid_spec,
        out_shape=jax.ShapeDtypeStruct((CAP, DIM), jnp.float32),
        compiler_params=pltpu.CompilerParams(
            dimension_semantics=("arbitrary", "arbitrary")),
    )(ntl, tstart, fetch_e, xs, W1, b1[:, None, :], W2, b2[:, None, :], wcol)


def kernel(x, Wg, bg, W1, b1, W2, b2):
    i32 = jnp.int32
    top1_2d, prob_2d = _gate(x, Wg, bg)
    top1 = top1_2d[:, 0]
    gp = prob_2d[:, 0]

    oh = (top1[:, None] == jnp.arange(E, dtype=i32)[None, :]).astype(i32)
    cum = jnp.cumsum(oh, axis=0)
    rank = jnp.take_along_axis(cum, top1[:, None], axis=1)[:, 0] - 1
    counts = cum[-1]
    ntl = (counts + TM - 1) // TM
    tstart = jnp.concatenate([jnp.zeros((1,), i32),
                              jnp.cumsum(ntl)[:-1].astype(i32)])
    slot = tstart[top1] * TM + rank
    tok = jnp.arange(TOKENS, dtype=i32)
    payload = jnp.stack([tok.astype(jnp.float32), gp], axis=1)
    packed = jnp.zeros((CAP, 2), jnp.float32).at[slot].set(payload)
    wcol = packed[:, 1:2]
    scat = jnp.where(packed[:, 1] > 0, packed[:, 0].astype(i32),
                     TOKENS + jnp.arange(CAP, dtype=i32))
    fe_raw = jnp.where(ntl > 0, jnp.arange(E, dtype=i32), -1)
    fetch_e = jnp.maximum(lax.cummax(fe_raw), 0).astype(i32)

    xs = _dispatch_rows(x, slot.astype(i32))
    ys = _grouped_ffn(xs, W1, b1, W2, b2, wcol,
                      ntl.astype(i32), tstart.astype(i32), fetch_e)
    out_big = _scatter_rows(ys, scat)
    return out_big[:TOKENS]


# speedup vs baseline: 2.4305x; 1.0553x over previous
import jax
import jax.numpy as jnp
from jax import lax
from jax.experimental import pallas as pl
from jax.experimental.pallas import tpu as pltpu
from jax.experimental.pallas import tpu_sc as plsc

TOKENS, DIM, HID, E = 2048, 1024, 4096, 8
TM = 128
NT = 24
CAP = NT * TM
TH = 2048
NH = HID // TH
NC, NS = 2, 16
NW = NC * NS
BPW = CAP // NW
BPW_IN = TOKENS // NW

_NEG = -1e30


def _gate_body(x_ref, wg_ref, bg_ref, top1_ref, prob_ref):
    l = jnp.dot(x_ref[...], wg_ref[...],
                preferred_element_type=jnp.float32) + bg_ref[...]
    m = jnp.max(l, axis=1, keepdims=True)
    s = jnp.sum(jnp.exp(l - m), axis=1, keepdims=True)
    col = lax.broadcasted_iota(jnp.int32, l.shape, 1)
    top1_ref[...] = jnp.min(jnp.where(l == m, col, 128), axis=1, keepdims=True)
    prob_ref[...] = 1.0 / s


def _gate(x, Wg, bg):
    wgp = jnp.pad(Wg, ((0, 0), (0, 128 - E)))
    bgp = jnp.pad(bg, (0, 128 - E), constant_values=_NEG)[None, :]
    return pl.pallas_call(
        _gate_body,
        out_shape=(jax.ShapeDtypeStruct((TOKENS, 1), jnp.int32),
                   jax.ShapeDtypeStruct((TOKENS, 1), jnp.float32)),
    )(x, wgp, bgp)


def _sc_scatter_in_body(x_hbm, slot_hbm, xs_hbm, idx_v, rows_v, sem):
    wid = lax.axis_index("s") * NC + lax.axis_index("c")
    base = wid * BPW_IN
    pltpu.sync_copy(slot_hbm.at[pl.ds(base, BPW_IN)], idx_v)
    pltpu.sync_copy(x_hbm.at[pl.ds(base, BPW_IN)], rows_v)
    pltpu.async_copy(rows_v, xs_hbm.at[idx_v], sem).wait()


def _dispatch_rows(x, slot):
    return pl.kernel(
        _sc_scatter_in_body,
        out_type=jax.ShapeDtypeStruct((CAP, DIM), jnp.float32),
        mesh=plsc.VectorSubcoreMesh(core_axis_name="c", subcore_axis_name="s"),
        scratch_types=[pltpu.VMEM((BPW_IN,), jnp.int32),
                       pltpu.VMEM((BPW_IN, DIM), jnp.float32),
                       pltpu.SemaphoreType.DMA],
    )(x, slot)


_CK = 8
_NCH = BPW_IN // _CK


def _sc_collect_body(ys_hbm, slot_hbm, out_hbm, idx_v, rows_v, sem):
    wid = lax.axis_index("s") * NC + lax.axis_index("c")
    base = wid * BPW_IN
    pltpu.sync_copy(slot_hbm.at[pl.ds(base, BPW_IN)], idx_v)
    descs = []
    for j in range(_NCH):
        descs.append(pltpu.async_copy(
            ys_hbm.at[idx_v.at[pl.ds(j * _CK, _CK)]],
            rows_v.at[pl.ds(j * _CK, _CK)], sem))
    for d in descs:
        d.wait()
    pltpu.sync_copy(rows_v, out_hbm.at[pl.ds(base, BPW_IN)])


def _collect_rows(ys, slot):
    return pl.kernel(
        _sc_collect_body,
        out_type=jax.ShapeDtypeStruct((TOKENS, DIM), jnp.float32),
        mesh=plsc.VectorSubcoreMesh(core_axis_name="c", subcore_axis_name="s"),
        scratch_types=[pltpu.VMEM((BPW_IN,), jnp.int32),
                       pltpu.VMEM((BPW_IN, DIM), jnp.float32),
                       pltpu.SemaphoreType.DMA],
    )(ys, slot)


def _ffn_body(ntl_ref, ts_ref, fe_ref, xs_ref, w1_ref, b1_ref, w2_ref, b2_ref,
              wv_ref, out_ref):
    del fe_ref
    e = pl.program_id(0)
    h = pl.program_id(1)
    ne = ntl_ref[e]
    ts = ts_ref[e]

    def tile_body(i, carry):
        base = (ts + i) * TM
        rows = pl.ds(base, TM)
        xt = xs_ref[rows, :]
        ht = jnp.maximum(
            jnp.dot(xt, w1_ref[0], preferred_element_type=jnp.float32)
            + b1_ref[0], 0.0)
        yt = jnp.dot(ht, w2_ref[0], preferred_element_type=jnp.float32)

        @pl.when(h == 0)
        def _():
            out_ref[rows, :] = yt

        @pl.when(h != 0)
        def _():
            out_ref[rows, :] = out_ref[rows, :] + yt

        @pl.when(h == NH - 1)
        def _():
            wt = wv_ref[0, rows]
            wb = jnp.broadcast_to(wt[None, :], (TM, TM))
            ri = lax.broadcasted_iota(jnp.int32, (TM, TM), 0)
            ci = lax.broadcasted_iota(jnp.int32, (TM, TM), 1)
            dw = jnp.where(ri == ci, wb, 0.0)
            m = out_ref[rows, :] + b2_ref[0]
            m = jnp.where(jnp.abs(m) <= 3.0e38, m, 0.0)
            out_ref[rows, :] = jnp.dot(
                dw, m,
                preferred_element_type=jnp.float32,
                precision=lax.Precision.HIGHEST)

        return carry

    lax.fori_loop(0, ne, tile_body, 0)


def _grouped_ffn(xs, W1, b1, W2, b2, wcol, ntl, tstart, fetch_e):
    grid_spec = pltpu.PrefetchScalarGridSpec(
        num_scalar_prefetch=3,
        grid=(E, NH),
        in_specs=[
            pl.BlockSpec((CAP, DIM), lambda e, h, ntl, ts, fe: (0, 0)),
            pl.BlockSpec((1, DIM, TH), lambda e, h, ntl, ts, fe: (fe[e], 0, h)),
            pl.BlockSpec((1, 1, TH), lambda e, h, ntl, ts, fe: (fe[e], 0, h)),
            pl.BlockSpec((1, TH, DIM), lambda e, h, ntl, ts, fe: (fe[e], h, 0)),
            pl.BlockSpec((1, 1, DIM), lambda e, h, ntl, ts, fe: (fe[e], 0, 0)),
            pl.BlockSpec((1, CAP), lambda e, h, ntl, ts, fe: (0, 0)),
        ],
        out_specs=pl.BlockSpec((CAP, DIM), lambda e, h, ntl, ts, fe: (0, 0)),
    )
    return pl.pallas_call(
        _ffn_body,
        grid_spec=grid_spec,
        out_shape=jax.ShapeDtypeStruct((CAP, DIM), jnp.float32),
        compiler_params=pltpu.CompilerParams(
            dimension_semantics=("arbitrary", "arbitrary")),
    )(ntl, tstart, fetch_e, xs, W1, b1[:, None, :], W2, b2[:, None, :], wcol)


def kernel(x, Wg, bg, W1, b1, W2, b2):
    i32 = jnp.int32
    top1_2d, prob_2d = _gate(x, Wg, bg)
    top1 = top1_2d[:, 0]
    gp = prob_2d[:, 0]

    oh = (top1[:, None] == jnp.arange(E, dtype=i32)[None, :]).astype(i32)
    cum = jnp.cumsum(oh, axis=0)
    rank = jnp.take_along_axis(cum, top1[:, None], axis=1)[:, 0] - 1
    counts = cum[-1]
    ntl = (counts + TM - 1) // TM
    tstart = jnp.concatenate([jnp.zeros((1,), i32),
                              jnp.cumsum(ntl)[:-1].astype(i32)])
    slot = tstart[top1] * TM + rank
    wrow = jnp.zeros((CAP,), jnp.float32).at[slot].set(gp)[None, :]
    fe_raw = jnp.where(ntl > 0, jnp.arange(E, dtype=i32), -1)
    fetch_e = jnp.maximum(lax.cummax(fe_raw), 0).astype(i32)

    slot = slot.astype(i32)
    xs = _dispatch_rows(x, slot)
    ys = _grouped_ffn(xs, W1, b1, W2, b2, wrow,
                      ntl.astype(i32), tstart.astype(i32), fetch_e)
    return _collect_rows(ys, slot)


# speedup vs baseline: 2.6125x; 1.0749x over previous
import jax
import jax.numpy as jnp
from jax import lax
from jax.experimental import pallas as pl
from jax.experimental.pallas import tpu as pltpu
from jax.experimental.pallas import tpu_sc as plsc

TOKENS, DIM, HID, E = 2048, 1024, 4096, 8
TM = 128
NT = 24
CAP = NT * TM
TH = 2048
NH = HID // TH
NC, NS = 2, 16
NW = NC * NS
BPW = CAP // NW
BPW_IN = TOKENS // NW

_NEG = -1e30


def _gate_body(x_ref, wg_ref, bg_ref, top1_ref, prob_ref):
    l = jnp.dot(x_ref[...], wg_ref[...],
                preferred_element_type=jnp.float32) + bg_ref[...]
    m = jnp.max(l, axis=1, keepdims=True)
    s = jnp.sum(jnp.exp(l - m), axis=1, keepdims=True)
    col = lax.broadcasted_iota(jnp.int32, l.shape, 1)
    top1_ref[...] = jnp.min(jnp.where(l == m, col, 128), axis=1, keepdims=True)
    prob_ref[...] = 1.0 / s


def _gate(x, Wg, bg):
    wgp = jnp.pad(Wg, ((0, 0), (0, 128 - E)))
    bgp = jnp.pad(bg, (0, 128 - E), constant_values=_NEG)[None, :]
    return pl.pallas_call(
        _gate_body,
        out_shape=(jax.ShapeDtypeStruct((TOKENS, 1), jnp.int32),
                   jax.ShapeDtypeStruct((TOKENS, 1), jnp.float32)),
    )(x, wgp, bgp)


def _sc_scatter_in_body(x_hbm, slot_hbm, xs_hbm, idx_v, rows_v, sem):
    wid = lax.axis_index("s") * NC + lax.axis_index("c")
    base = wid * BPW_IN
    pltpu.sync_copy(slot_hbm.at[pl.ds(base, BPW_IN)], idx_v)
    pltpu.sync_copy(x_hbm.at[pl.ds(base, BPW_IN)], rows_v)
    pltpu.async_copy(rows_v, xs_hbm.at[idx_v], sem).wait()


def _dispatch_rows(x, slot):
    return pl.kernel(
        _sc_scatter_in_body,
        out_type=jax.ShapeDtypeStruct((CAP, DIM), jnp.float32),
        mesh=plsc.VectorSubcoreMesh(core_axis_name="c", subcore_axis_name="s"),
        scratch_types=[pltpu.VMEM((BPW_IN,), jnp.int32),
                       pltpu.VMEM((BPW_IN, DIM), jnp.float32),
                       pltpu.SemaphoreType.DMA],
    )(x, slot)


_CK = 8
_NCH = BPW_IN // _CK


def _sc_collect_body(ys_hbm, slot_hbm, out_hbm, idx_v, rows_v, sem):
    wid = lax.axis_index("s") * NC + lax.axis_index("c")
    base = wid * BPW_IN
    pltpu.sync_copy(slot_hbm.at[pl.ds(base, BPW_IN)], idx_v)
    descs = []
    for j in range(_NCH):
        descs.append(pltpu.async_copy(
            ys_hbm.at[idx_v.at[pl.ds(j * _CK, _CK)]],
            rows_v.at[pl.ds(j * _CK, _CK)], sem))
    for d in descs:
        d.wait()
    pltpu.sync_copy(rows_v, out_hbm.at[pl.ds(base, BPW_IN)])


def _collect_rows(ys, slot):
    return pl.kernel(
        _sc_collect_body,
        out_type=jax.ShapeDtypeStruct((TOKENS, DIM), jnp.float32),
        mesh=plsc.VectorSubcoreMesh(core_axis_name="c", subcore_axis_name="s"),
        scratch_types=[pltpu.VMEM((BPW_IN,), jnp.int32),
                       pltpu.VMEM((BPW_IN, DIM), jnp.float32),
                       pltpu.SemaphoreType.DMA],
    )(ys, slot)


def _ffn_body(ntl_ref, ts_ref, fe_ref, xs_ref, w1_ref, b1_ref, w2_ref, b2_ref,
              wv_ref, out_ref):
    del fe_ref
    e = pl.program_id(0)
    h = pl.program_id(1)
    ne = ntl_ref[e]
    ts = ts_ref[e]

    def tile_body(i, carry):
        base = (ts + i) * TM
        rows = pl.ds(base, TM)
        xt = xs_ref[rows, :].astype(jnp.bfloat16)
        ht = jnp.maximum(
            jnp.dot(xt, w1_ref[0].astype(jnp.bfloat16),
                    preferred_element_type=jnp.float32)
            + b1_ref[0], 0.0)
        yt = jnp.dot(ht.astype(jnp.bfloat16), w2_ref[0].astype(jnp.bfloat16),
                     preferred_element_type=jnp.float32)

        @pl.when(h == 0)
        def _():
            out_ref[rows, :] = yt

        @pl.when(h != 0)
        def _():
            out_ref[rows, :] = out_ref[rows, :] + yt

        @pl.when(h == NH - 1)
        def _():
            wt = wv_ref[0, rows]
            wb = jnp.broadcast_to(wt[None, :], (TM, TM))
            ri = lax.broadcasted_iota(jnp.int32, (TM, TM), 0)
            ci = lax.broadcasted_iota(jnp.int32, (TM, TM), 1)
            dw = jnp.where(ri == ci, wb, 0.0)
            m = out_ref[rows, :] + b2_ref[0]
            m = jnp.where(jnp.abs(m) <= 3.0e38, m, 0.0)
            out_ref[rows, :] = jnp.dot(
                dw, m,
                preferred_element_type=jnp.float32,
                precision=lax.Precision.HIGHEST)

        return carry

    lax.fori_loop(0, ne, tile_body, 0)


def _grouped_ffn(xs, W1, b1, W2, b2, wcol, ntl, tstart, fetch_e):
    grid_spec = pltpu.PrefetchScalarGridSpec(
        num_scalar_prefetch=3,
        grid=(E, NH),
        in_specs=[
            pl.BlockSpec((CAP, DIM), lambda e, h, ntl, ts, fe: (0, 0)),
            pl.BlockSpec((1, DIM, TH), lambda e, h, ntl, ts, fe: (fe[e], 0, h)),
            pl.BlockSpec((1, 1, TH), lambda e, h, ntl, ts, fe: (fe[e], 0, h)),
            pl.BlockSpec((1, TH, DIM), lambda e, h, ntl, ts, fe: (fe[e], h, 0)),
            pl.BlockSpec((1, 1, DIM), lambda e, h, ntl, ts, fe: (fe[e], 0, 0)),
            pl.BlockSpec((1, CAP), lambda e, h, ntl, ts, fe: (0, 0)),
        ],
        out_specs=pl.BlockSpec((CAP, DIM), lambda e, h, ntl, ts, fe: (0, 0)),
    )
    return pl.pallas_call(
        _ffn_body,
        grid_spec=grid_spec,
        out_shape=jax.ShapeDtypeStruct((CAP, DIM), jnp.float32),
        compiler_params=pltpu.CompilerParams(
            dimension_semantics=("arbitrary", "arbitrary")),
    )(ntl, tstart, fetch_e, xs, W1, b1[:, None, :], W2, b2[:, None, :], wcol)


def kernel(x, Wg, bg, W1, b1, W2, b2):
    i32 = jnp.int32
    top1_2d, prob_2d = _gate(x, Wg, bg)
    top1 = top1_2d[:, 0]
    gp = prob_2d[:, 0]

    oh = (top1[:, None] == jnp.arange(E, dtype=i32)[None, :]).astype(i32)
    cum = jnp.cumsum(oh, axis=0)
    rank = jnp.sum(cum * oh, axis=1) - 1
    counts = cum[-1]
    ntl = (counts + TM - 1) // TM
    tstart = jnp.concatenate([jnp.zeros((1,), i32),
                              jnp.cumsum(ntl)[:-1].astype(i32)])
    pad_base = jnp.sum((tstart * TM)[None, :] * oh, axis=1)
    slot = pad_base + rank
    wrow = jnp.zeros((CAP,), jnp.float32).at[slot].set(
        gp, unique_indices=True)[None, :]
    fe_raw = jnp.where(ntl > 0, jnp.arange(E, dtype=i32), -1)
    fetch_e = jnp.maximum(lax.cummax(fe_raw), 0).astype(i32)

    slot = slot.astype(i32)
    xs = _dispatch_rows(x, slot)
    ys = _grouped_ffn(xs, W1, b1, W2, b2, wrow,
                      ntl.astype(i32), tstart.astype(i32), fetch_e)
    return _collect_rows(ys, slot)


# speedup vs baseline: 2.6779x; 1.0250x over previous
import jax
import jax.numpy as jnp
from jax import lax
from jax.experimental import pallas as pl
from jax.experimental.pallas import tpu as pltpu
from jax.experimental.pallas import tpu_sc as plsc

TOKENS, DIM, HID, E = 2048, 1024, 4096, 8
TM = 128
NT = 24
CAP = NT * TM
TH = 2048
NH = HID // TH
NC, NS = 2, 16
NW = NC * NS
BPW = CAP // NW
BPW_IN = TOKENS // NW


def _gate_body(x_ref, wg_ref, bg_ref, top1_ref, prob_ref):
    l = jnp.dot(x_ref[...], wg_ref[...],
                preferred_element_type=jnp.float32) + bg_ref[...]
    m = jnp.max(l, axis=1, keepdims=True)
    s = jnp.sum(jnp.exp(l - m), axis=1, keepdims=True)
    col = lax.broadcasted_iota(jnp.int32, l.shape, 1)
    top1_ref[...] = jnp.min(jnp.where(l == m, col, E), axis=1, keepdims=True)
    prob_ref[...] = 1.0 / s


def _gate(x, Wg, bg):
    return pl.pallas_call(
        _gate_body,
        out_shape=(jax.ShapeDtypeStruct((TOKENS, 1), jnp.int32),
                   jax.ShapeDtypeStruct((TOKENS, 1), jnp.float32)),
    )(x, Wg, bg[None, :])


def _sc_scatter_in_body(x_hbm, slot_hbm, xs_hbm, idx_v, rows_v, sem):
    wid = lax.axis_index("s") * NC + lax.axis_index("c")
    base = wid * BPW_IN
    pltpu.sync_copy(slot_hbm.at[pl.ds(base, BPW_IN)], idx_v)
    pltpu.sync_copy(x_hbm.at[pl.ds(base, BPW_IN)], rows_v)
    pltpu.async_copy(rows_v, xs_hbm.at[idx_v], sem).wait()


def _dispatch_rows(x, slot):
    return pl.kernel(
        _sc_scatter_in_body,
        out_type=jax.ShapeDtypeStruct((CAP, DIM), jnp.float32),
        mesh=plsc.VectorSubcoreMesh(core_axis_name="c", subcore_axis_name="s"),
        scratch_types=[pltpu.VMEM((BPW_IN,), jnp.int32),
                       pltpu.VMEM((BPW_IN, DIM), jnp.float32),
                       pltpu.SemaphoreType.DMA],
    )(x, slot)


_CK = 8
_NCH = BPW_IN // _CK


def _sc_collect_body(ys_hbm, slot_hbm, out_hbm, idx_v, rows_v, sem):
    wid = lax.axis_index("s") * NC + lax.axis_index("c")
    base = wid * BPW_IN
    pltpu.sync_copy(slot_hbm.at[pl.ds(base, BPW_IN)], idx_v)
    descs = []
    for j in range(_NCH):
        descs.append(pltpu.async_copy(
            ys_hbm.at[idx_v.at[pl.ds(j * _CK, _CK)]],
            rows_v.at[pl.ds(j * _CK, _CK)], sem))
    for d in descs:
        d.wait()
    pltpu.sync_copy(rows_v, out_hbm.at[pl.ds(base, BPW_IN)])


def _collect_rows(ys, slot):
    return pl.kernel(
        _sc_collect_body,
        out_type=jax.ShapeDtypeStruct((TOKENS, DIM), jnp.float32),
        mesh=plsc.VectorSubcoreMesh(core_axis_name="c", subcore_axis_name="s"),
        scratch_types=[pltpu.VMEM((BPW_IN,), jnp.int32),
                       pltpu.VMEM((BPW_IN, DIM), jnp.float32),
                       pltpu.SemaphoreType.DMA],
    )(ys, slot)


def _ffn_body(ntl_ref, ts_ref, fe_ref, xs_ref, w1_ref, b1_ref, w2_ref, b2_ref,
              wv_ref, out_ref):
    del fe_ref
    e = pl.program_id(0)
    h = pl.program_id(1)
    ne = ntl_ref[e]
    ts = ts_ref[e]

    def tile_body(i, carry):
        base = (ts + i) * TM
        rows = pl.ds(base, TM)
        xt = xs_ref[rows, :]
        ht = jnp.maximum(
            jnp.dot(xt, w1_ref[0], preferred_element_type=jnp.float32)
            + b1_ref[0], 0.0)
        yt = jnp.dot(ht, w2_ref[0], preferred_element_type=jnp.float32)

        @pl.when(h == 0)
        def _():
            out_ref[rows, :] = yt

        @pl.when(h != 0)
        def _():
            out_ref[rows, :] = out_ref[rows, :] + yt

        @pl.when(h == NH - 1)
        def _():
            wt = wv_ref[0, rows]
            wb = jnp.broadcast_to(wt[None, :], (TM, TM))
            ri = lax.broadcasted_iota(jnp.int32, (TM, TM), 0)
            ci = lax.broadcasted_iota(jnp.int32, (TM, TM), 1)
            dw = jnp.where(ri == ci, wb, 0.0)
            m = out_ref[rows, :] + b2_ref[0]
            m = jnp.where(jnp.abs(m) <= 3.0e38, m, 0.0)
            out_ref[rows, :] = jnp.dot(
                dw, m,
                preferred_element_type=jnp.float32,
                precision=lax.Precision.HIGHEST)

        return carry

    lax.fori_loop(0, ne, tile_body, 0)


def _grouped_ffn(xs, W1, b1, W2, b2, wcol, ntl, tstart, fetch_e):
    grid_spec = pltpu.PrefetchScalarGridSpec(
        num_scalar_prefetch=3,
        grid=(E, NH),
        in_specs=[
            pl.BlockSpec((CAP, DIM), lambda e, h, ntl, ts, fe: (0, 0)),
            pl.BlockSpec((1, DIM, TH), lambda e, h, ntl, ts, fe: (fe[e], 0, h)),
            pl.BlockSpec((1, 1, TH), lambda e, h, ntl, ts, fe: (fe[e], 0, h)),
            pl.BlockSpec((1, TH, DIM), lambda e, h, ntl, ts, fe: (fe[e], h, 0)),
            pl.BlockSpec((1, 1, DIM), lambda e, h, ntl, ts, fe: (fe[e], 0, 0)),
            pl.BlockSpec((1, CAP), lambda e, h, ntl, ts, fe: (0, 0)),
        ],
        out_specs=pl.BlockSpec((CAP, DIM), lambda e, h, ntl, ts, fe: (0, 0)),
    )
    return pl.pallas_call(
        _ffn_body,
        grid_spec=grid_spec,
        out_shape=jax.ShapeDtypeStruct((CAP, DIM), jnp.float32),
        compiler_params=pltpu.CompilerParams(
            dimension_semantics=("arbitrary", "arbitrary")),
    )(ntl, tstart, fetch_e, xs, W1, b1[:, None, :], W2, b2[:, None, :], wcol)


def kernel(x, Wg, bg, W1, b1, W2, b2):
    i32 = jnp.int32
    top1_2d, prob_2d = _gate(x, Wg, bg)
    top1 = top1_2d[:, 0]
    gp = prob_2d[:, 0]

    oh = (top1[:, None] == jnp.arange(E, dtype=i32)[None, :]).astype(i32)
    cum = jnp.cumsum(oh, axis=0)
    rank = jnp.sum(cum * oh, axis=1) - 1
    counts = cum[-1]
    ntl = (counts + TM - 1) // TM
    tstart = jnp.concatenate([jnp.zeros((1,), i32),
                              jnp.cumsum(ntl)[:-1].astype(i32)])
    pad_base = jnp.sum((tstart * TM)[None, :] * oh, axis=1)
    slot = pad_base + rank
    wrow = jnp.zeros((CAP,), jnp.float32).at[slot].set(
        gp, unique_indices=True)[None, :]
    fe_raw = jnp.where(ntl > 0, jnp.arange(E, dtype=i32), -1)
    fetch_e = jnp.maximum(lax.cummax(fe_raw), 0).astype(i32)

    slot = slot.astype(i32)
    xs = _dispatch_rows(x, slot)
    ys = _grouped_ffn(xs, W1, b1, W2, b2, wrow,
                      ntl.astype(i32), tstart.astype(i32), fetch_e)
    return _collect_rows(ys, slot)
